# baseline (speedup 1.0000x reference)
import functools

import jax
import jax.numpy as jnp
from jax import lax
from jax.experimental import pallas as pl
from jax.experimental.pallas import tpu as pltpu
from jax.experimental.pallas import tpu_sc as plsc

B = 16384
NC = 26
NN = 13
V = 100000
D = 32
NF = NC + NN

_NCORES = 2
_NSUB = 16
_NW = _NCORES * _NSUB
_ROWS_PER_W = B // _NW
_G = 8
_NGRP = _ROWS_PER_W // _G
_NBUF = 2
_OUTER = _NGRP // _NBUF


def _sc_body(idx_hbm, xnum_hbm, table_hbm, w_hbm, b_hbm, out_hbm,
             idxbuf, xbuf, wv, bv, obuf, gsem, osem0, osem1):
    osems = (osem0, osem1)
    wid = lax.axis_index("s") * _NCORES + lax.axis_index("c")
    wbase = wid * _ROWS_PER_W

    pltpu.sync_copy(w_hbm, wv)
    pltpu.sync_copy(b_hbm, bv)

    def outer_body(outer, carry):
        for slot in range(_NBUF):
            g = outer * _NBUF + slot
            b0 = wbase + g * _G

            @pl.when(outer >= 1)
            def _drain_prev():
                pltpu.make_async_copy(
                    out_hbm.at[pl.ds(0, _G)], obuf.at[slot], osems[slot]
                ).wait()

            pltpu.sync_copy(idx_hbm.at[pl.ds(b0, _G)], idxbuf.at[slot])
            pltpu.sync_copy(xnum_hbm.at[pl.ds(b0, _G)], xbuf.at[slot])

            cps = [
                pltpu.async_copy(
                    table_hbm.at[idxbuf.at[slot, c]],
                    obuf.at[slot, c, pl.ds(0, NC)],
                    gsem,
                )
                for c in range(_G)
            ]

            def num_body(c, _):
                xv = xbuf[slot, c, pl.ds(0, 16)]
                for j in range(NN):
                    x = xv[j]
                    for h in range(D // 16):
                        wrow = wv[j, pl.ds(h * 16, 16)]
                        brow = bv[j, pl.ds(h * 16, 16)]
                        obuf[slot, c, NC + j, pl.ds(h * 16, 16)] = x * wrow + brow
                return 0

            lax.fori_loop(0, _G, num_body, 0)

            for cp in cps:
                cp.wait()

            pltpu.async_copy(obuf.at[slot], out_hbm.at[pl.ds(b0, _G)], osems[slot])
        return carry

    lax.fori_loop(0, _OUTER, outer_body, 0)

    for slot in range(_NBUF):
        pltpu.make_async_copy(
            out_hbm.at[pl.ds(0, _G)], obuf.at[slot], osems[slot]
        ).wait()


@functools.partial(jax.jit, static_argnums=())
def _tokenize(idx_flat, x_num, table_flat, num_W, num_b):
    mesh = plsc.VectorSubcoreMesh(core_axis_name="c", subcore_axis_name="s")
    kern = functools.partial(
        pl.kernel,
        out_type=jax.ShapeDtypeStruct((B, NF, D), jnp.float32),
        mesh=mesh,
        scratch_types=[
            pltpu.VMEM((_NBUF, _G, NC), jnp.int32),
            pltpu.VMEM((_NBUF, _G, 16), jnp.float32),
            pltpu.VMEM((NN, D), jnp.float32),
            pltpu.VMEM((NN, D), jnp.float32),
            pltpu.VMEM((_NBUF, _G, NF, D), jnp.float32),
            pltpu.SemaphoreType.DMA,
            pltpu.SemaphoreType.DMA,
            pltpu.SemaphoreType.DMA,
        ],
        compiler_params=pltpu.CompilerParams(use_tc_tiling_on_sc=False),
    )(_sc_body)
    return kern(idx_flat, x_num, table_flat, num_W, num_b)


def kernel(x_num, x_cat, cat_tables, num_W, num_b):
    offs = (jnp.arange(NC, dtype=jnp.int32) * V)[None, :]
    idx_flat = x_cat.astype(jnp.int32) + offs
    table_flat = cat_tables.reshape(NC * V, D)
    x_pad = jnp.pad(x_num, ((0, 0), (0, 16 - NN)))
    return _tokenize(idx_flat, x_pad, table_flat, num_W, num_b)


# speedup vs baseline: 1.0204x; 1.0204x over previous
import functools

import jax
import jax.numpy as jnp
from jax import lax
from jax.experimental import pallas as pl
from jax.experimental.pallas import tpu as pltpu
from jax.experimental.pallas import tpu_sc as plsc

B = 16384
NC = 26
NN = 13
V = 100000
D = 32
NF = NC + NN

_NCORES = 2
_NSUB = 16
_NW = _NCORES * _NSUB
_ROWS_PER_W = B // _NW
_G = 8
_NGRP = _ROWS_PER_W // _G
_SPG = (_G * NC) // 104


def _sc_body(idx_hbm, xnum_hbm, table_hbm, w_hbm, b_hbm, out_hbm,
             idxbuf, xbuf, wv, bv, catbuf, numbuf, dbuf,
             gsem0, gsem1, osem0, osem1, isem):
    gsems = (gsem0, gsem1)
    osems = (osem0, osem1)
    wid = lax.axis_index("s") * _NCORES + lax.axis_index("c")
    wbase = wid * _ROWS_PER_W

    pltpu.sync_copy(w_hbm, wv)
    pltpu.sync_copy(b_hbm, bv)

    def load_inputs(g, slot, sync):
        b0 = wbase + g * _G
        copies = (
            (idx_hbm.at[pl.ds(b0 * NC // 104, _SPG)], idxbuf.at[slot]),
            (xnum_hbm.at[pl.ds(b0, _G)], xbuf.at[slot]),
        )
        for src, dst in copies:
            if sync:
                pltpu.sync_copy(src, dst)
            else:
                pltpu.async_copy(src, dst, isem)

    def wait_inputs(slot):
        pltpu.make_async_copy(
            idx_hbm.at[pl.ds(0, _SPG)], idxbuf.at[slot], isem).wait()
        pltpu.make_async_copy(
            xnum_hbm.at[pl.ds(0, _G)], xbuf.at[slot], isem).wait()

    def fire_gathers(slot):
        for k in range(_SPG):
            pltpu.async_copy(
                table_hbm.at[idxbuf.at[slot, k]],
                catbuf.at[slot, pl.ds(k * 104, 104)],
                gsems[slot],
            )

    def drain_gathers(slot):
        pltpu.make_async_copy(
            table_hbm.at[pl.ds(0, _G * NC)], catbuf.at[slot], gsems[slot]
        ).wait()

    def compute_num(slot):
        def num_body(c, carry):
            xv = xbuf[slot, c, pl.ds(0, 16)]
            for j in range(NN):
                x = xv[j]
                for h in range(D // 16):
                    wrow = wv[j, pl.ds(h * 16, 16)]
                    brow = bv[j, pl.ds(h * 16, 16)]
                    numbuf[slot, c, j, pl.ds(h * 16, 16)] = x * wrow + brow
            return carry
        lax.fori_loop(0, _G, num_body, 0)

    def fire_outs(g, slot):
        b0 = wbase + g * _G
        for c in range(_G):
            pltpu.async_copy(
                catbuf.at[slot, pl.ds(c * NC, NC)],
                out_hbm.at[b0 + c, pl.ds(0, NC)],
                osems[slot],
            )
            pltpu.async_copy(
                numbuf.at[slot, c],
                out_hbm.at[b0 + c, pl.ds(NC, NN)],
                osems[slot],
            )

    def drain_outs(slot):
        pltpu.make_async_copy(
            out_hbm.at[pl.ds(0, _G)], dbuf, osems[slot]).wait()

    def outer_body(outer, carry):
        for slot in range(2):
            g = outer * 2 + slot
            other = 1 - slot

            @pl.when(g >= 2)
            def _a():
                drain_outs(slot)

            @pl.when(g == 0)
            def _b0():
                load_inputs(g, slot, sync=True)

            @pl.when(g >= 1)
            def _b1():
                wait_inputs(slot)

            fire_gathers(slot)
            compute_num(slot)

            @pl.when(g >= 1)
            def _e():
                drain_gathers(other)

            @pl.when(g < _NGRP - 1)
            def _f():
                load_inputs(g + 1, other, sync=False)

            @pl.when(g >= 1)
            def _h():
                fire_outs(g - 1, other)
        return carry

    lax.fori_loop(0, _NGRP // 2, outer_body, 0)

    drain_gathers(1)
    fire_outs(_NGRP - 1, 1)
    drain_outs(0)
    drain_outs(1)


@jax.jit
def _tokenize(idx_flat, x_num, table_flat, num_W, num_b):
    mesh = plsc.VectorSubcoreMesh(core_axis_name="c", subcore_axis_name="s")
    kern = functools.partial(
        pl.kernel,
        out_type=jax.ShapeDtypeStruct((B, NF, D), jnp.float32),
        mesh=mesh,
        scratch_types=[
            pltpu.VMEM((2, _SPG, 104), jnp.int32),
            pltpu.VMEM((2, _G, 16), jnp.float32),
            pltpu.VMEM((NN, D), jnp.float32),
            pltpu.VMEM((NN, D), jnp.float32),
            pltpu.VMEM((2, _G * NC, D), jnp.float32),
            pltpu.VMEM((2, _G, NN, D), jnp.float32),
            pltpu.VMEM((_G, NF, D), jnp.float32),
            pltpu.SemaphoreType.DMA,
            pltpu.SemaphoreType.DMA,
            pltpu.SemaphoreType.DMA,
            pltpu.SemaphoreType.DMA,
            pltpu.SemaphoreType.DMA,
        ],
        compiler_params=pltpu.CompilerParams(use_tc_tiling_on_sc=False),
    )(_sc_body)
    return kern(idx_flat, x_num, table_flat, num_W, num_b)


def kernel(x_num, x_cat, cat_tables, num_W, num_b):
    offs = (jnp.arange(NC, dtype=jnp.int32) * V)[None, :]
    idx_flat = (x_cat.astype(jnp.int32) + offs).reshape(B * NC // 104, 104)
    table_flat = cat_tables.reshape(NC * V, D)
    x_pad = jnp.pad(x_num, ((0, 0), (0, 16 - NN)))
    return _tokenize(idx_flat, x_pad, table_flat, num_W, num_b)


# speedup vs baseline: 3.1052x; 3.0430x over previous
import functools

import jax
import jax.numpy as jnp
from jax import lax
from jax.experimental import pallas as pl
from jax.experimental.pallas import tpu as pltpu
from jax.experimental.pallas import tpu_sc as plsc

B = 16384
NC = 26
NN = 13
V = 100000
D = 32
NF = NC + NN

_NCORES = 2
_NSUB = 16
_NW = _NCORES * _NSUB
_BC = 4096
_NCHUNK = B // _BC


def _sc_body(xcat_hbm, xnum_hbm, table_hbm, wt_hbm, bt_hbm, out_hbm,
             planebuf, colbuf, obuf, xchunk, wtv, btv, psem, csem, osem):
    wid = lax.axis_index("s") * _NCORES + lax.axis_index("c")
    d = wid

    pltpu.sync_copy(wt_hbm, wtv)
    pltpu.sync_copy(bt_hbm, btv)
    wvec = wtv[d, pl.ds(0, 16)]
    bvec = btv[d, pl.ds(0, 16)]
    zidx = jnp.zeros((16,), jnp.int32)

    def do_cat_plane(f):
        pltpu.sync_copy(table_hbm.at[pl.ds(f, 1), pl.ds(d, 1), :], planebuf)
        for ch in range(_NCHUNK):
            b0 = ch * _BC
            pltpu.sync_copy(xcat_hbm.at[pl.ds(f, 1), pl.ds(b0, _BC)], colbuf)

            def gath(i, carry):
                iv = colbuf[0, pl.ds(i * 16, 16)]
                obuf[0, 0, pl.ds(i * 16, 16)] = plsc.load_gather(
                    planebuf, [zidx, zidx, iv])
                return carry
            lax.fori_loop(0, _BC // 16, gath, 0)
            pltpu.sync_copy(
                obuf,
                out_hbm.at[pl.ds(f, 1), pl.ds(d, 1), pl.ds(b0, _BC)])

    def do_num_plane(j):
        w_jd = wvec[j]
        b_jd = bvec[j]
        for ch in range(_NCHUNK):
            b0 = ch * _BC
            pltpu.sync_copy(xnum_hbm.at[pl.ds(j, 1), pl.ds(b0, _BC)], xchunk)

            def proj(i, carry):
                xv = xchunk[0, pl.ds(i * 16, 16)]
                obuf[0, 0, pl.ds(i * 16, 16)] = xv * w_jd + b_jd
                return carry
            lax.fori_loop(0, _BC // 16, proj, 0)
            pltpu.sync_copy(
                obuf,
                out_hbm.at[pl.ds(NC + j, 1), pl.ds(d, 1), pl.ds(b0, _BC)])

    for f in range(NC):
        do_cat_plane(f)
    for j in range(NN):
        do_num_plane(j)


@jax.jit
def _tokenize(x_catT, x_numT, tableT, wT, bT):
    mesh = plsc.VectorSubcoreMesh(core_axis_name="c", subcore_axis_name="s")
    kern = functools.partial(
        pl.kernel,
        out_type=jax.ShapeDtypeStruct((NF, D, B), jnp.float32),
        mesh=mesh,
        scratch_types=[
            pltpu.VMEM((1, 1, V), jnp.float32),
            pltpu.VMEM((1, _BC), jnp.int32),
            pltpu.VMEM((1, 1, _BC), jnp.float32),
            pltpu.VMEM((1, _BC), jnp.float32),
            pltpu.VMEM((D, 16), jnp.float32),
            pltpu.VMEM((D, 16), jnp.float32),
            pltpu.SemaphoreType.DMA,
            pltpu.SemaphoreType.DMA,
            pltpu.SemaphoreType.DMA,
        ],
        compiler_params=pltpu.CompilerParams(
            use_tc_tiling_on_sc=True, needs_layout_passes=False),
    )(_sc_body)
    return kern(x_catT, x_numT, tableT, wT, bT)


def kernel(x_num, x_cat, cat_tables, num_W, num_b):
    tableT = jnp.transpose(cat_tables, (0, 2, 1))
    x_catT = x_cat.T.astype(jnp.int32)
    x_numT = x_num.T
    wT = jnp.pad(num_W.T, ((0, 0), (0, 16 - NN)))
    bT = jnp.pad(num_b.T, ((0, 0), (0, 16 - NN)))
    outT = _tokenize(x_catT, x_numT, tableT, wT, bT)
    return jnp.transpose(outT, (2, 0, 1))


# speedup vs baseline: 5.2798x; 1.7003x over previous
import functools

import jax
import jax.numpy as jnp
from jax import lax
from jax.experimental import pallas as pl
from jax.experimental.pallas import tpu as pltpu
from jax.experimental.pallas import tpu_sc as plsc

B = 16384
NC = 26
NN = 13
V = 100000
D = 32
NF = NC + NN

_NCORES = 2
_NSUB = 16
_NW = _NCORES * _NSUB
_BC = 4096
_NCHUNK = B // _BC
_NUM_UNITS = [(j, ch) for j in range(NN) for ch in range(_NCHUNK)]


def _sc_body(xcat_hbm, xnum_hbm, table_hbm, wt_hbm, bt_hbm, out_hbm,
             planebuf, colbuf, obuf, xchunk, wtv, btv, psem, csem, osem):
    wid = lax.axis_index("s") * _NCORES + lax.axis_index("c")
    d = wid

    pltpu.sync_copy(wt_hbm, wtv)
    pltpu.sync_copy(bt_hbm, btv)
    wvec = wtv[d, pl.ds(0, 16)]
    bvec = btv[d, pl.ds(0, 16)]
    zidx = jnp.zeros((16,), jnp.int32)

    state = {"slot": 0, "pending": [False, False]}

    def acquire_obuf():
        s = state["slot"]
        state["slot"] = 1 - s
        if state["pending"][s]:
            pltpu.make_async_copy(
                out_hbm.at[pl.ds(0, 1), pl.ds(0, 1), pl.ds(0, _BC)],
                obuf.at[s], osem).wait()
        state["pending"][s] = True
        return s

    def store_out(s, feat, ch):
        pltpu.async_copy(
            obuf.at[s],
            out_hbm.at[pl.ds(feat, 1), pl.ds(d, 1), pl.ds(ch * _BC, _BC)],
            osem)

    def fire_col(f, ch, cs):
        pltpu.async_copy(
            xcat_hbm.at[pl.ds(f, 1), pl.ds(ch * _BC, _BC)], colbuf.at[cs],
            csem)

    def wait_col(cs):
        pltpu.make_async_copy(
            xcat_hbm.at[pl.ds(0, 1), pl.ds(0, _BC)], colbuf.at[cs],
            csem).wait()

    def num_unit(j, ch):
        pltpu.sync_copy(
            xnum_hbm.at[pl.ds(j, 1), pl.ds(ch * _BC, _BC)], xchunk)
        w_jd = wvec[j]
        b_jd = bvec[j]
        s = acquire_obuf()

        def proj(i, carry):
            for u in range(2):
                xv = xchunk[0, pl.ds(i * 32 + u * 16, 16)]
                obuf[s, 0, 0, pl.ds(i * 32 + u * 16, 16)] = xv * w_jd + b_jd
            return carry
        lax.fori_loop(0, _BC // 32, proj, 0)
        store_out(s, NC + j, ch)

    def gather_chunk(f, ch, cs):
        wait_col(cs)
        if ch + 1 < _NCHUNK:
            fire_col(f, ch + 1, 1 - cs)
        s = acquire_obuf()

        def gath(i, carry):
            for u in range(2):
                iv = colbuf[cs, 0, pl.ds(i * 32 + u * 16, 16)]
                obuf[s, 0, 0, pl.ds(i * 32 + u * 16, 16)] = plsc.load_gather(
                    planebuf, [zidx, zidx, iv])
            return carry
        lax.fori_loop(0, _BC // 32, gath, 0)
        store_out(s, f, ch)

    for f in range(NC):
        pltpu.async_copy(
            table_hbm.at[pl.ds(f, 1), pl.ds(d, 1), :], planebuf, psem)
        fire_col(f, 0, 0)
        for j, ch in _NUM_UNITS[2 * f:2 * f + 2]:
            num_unit(j, ch)
        pltpu.make_async_copy(
            table_hbm.at[pl.ds(f, 1), pl.ds(d, 1), :], planebuf, psem).wait()
        for ch in range(_NCHUNK):
            gather_chunk(f, ch, ch % 2)

    for j, ch in _NUM_UNITS[2 * NC:]:
        num_unit(j, ch)

    for s in range(2):
        if state["pending"][s]:
            pltpu.make_async_copy(
                out_hbm.at[pl.ds(0, 1), pl.ds(0, 1), pl.ds(0, _BC)],
                obuf.at[s], osem).wait()


@jax.jit
def _tokenize(x_catT, x_numT, tableT, wT, bT):
    mesh = plsc.VectorSubcoreMesh(core_axis_name="c", subcore_axis_name="s")
    kern = functools.partial(
        pl.kernel,
        out_type=jax.ShapeDtypeStruct((NF, D, B), jnp.float32),
        mesh=mesh,
        scratch_types=[
            pltpu.VMEM((1, 1, V), jnp.float32),
            pltpu.VMEM((2, 1, _BC), jnp.int32),
            pltpu.VMEM((2, 1, 1, _BC), jnp.float32),
            pltpu.VMEM((1, _BC), jnp.float32),
            pltpu.VMEM((D, 16), jnp.float32),
            pltpu.VMEM((D, 16), jnp.float32),
            pltpu.SemaphoreType.DMA,
            pltpu.SemaphoreType.DMA,
            pltpu.SemaphoreType.DMA,
        ],
        compiler_params=pltpu.CompilerParams(
            use_tc_tiling_on_sc=True, needs_layout_passes=False),
    )(_sc_body)
    return kern(x_catT, x_numT, tableT, wT, bT)


def kernel(x_num, x_cat, cat_tables, num_W, num_b):
    tableT = jnp.transpose(cat_tables, (0, 2, 1))
    x_catT = x_cat.T.astype(jnp.int32)
    x_numT = x_num.T
    wT = jnp.pad(num_W.T, ((0, 0), (0, 16 - NN)))
    bT = jnp.pad(num_b.T, ((0, 0), (0, 16 - NN)))
    outT = _tokenize(x_catT, x_numT, tableT, wT, bT)
    return jnp.transpose(outT, (2, 0, 1))
